# Initial kernel scaffold; baseline (speedup 1.0000x reference)
import jax
import jax.numpy as jnp
from jax.experimental import pallas as pl


def kernel(x, edge_attr, mask, front_initial, edge_index):
    raise NotImplementedError("write your pallas kernel here")


# speedup vs baseline: 65.2180x; 65.2180x over previous
import jax
import jax.numpy as jnp
from jax import lax
from jax.experimental import pallas as pl
from jax.experimental.pallas import tpu as pltpu
from jax.experimental.pallas import tpu_sc as plsc

N = 100000
E = 3200000
NC, NS, L = 2, 16, 16
NW = NC * NS
NPT = 3136
NP = NPT * NW
CHUNK = 2048
NVEC = CHUNK // L
PAD = 4 * CHUNK
EP = E + PAD
H = 0.1
DST_BITS = 17
DST_MASK = (1 << DST_BITS) - 1


def _iota():
  return lax.iota(jnp.int32, L)


def _sweep(e0, e1, node_lo, pk_hbm, ce_hbm, y_full, gm_v, pk_b, ce_b,
           sem_pk, sem_ce):
  e0a = (e0 >> 4) << 4
  nch = (e1 - e0a + (CHUNK - 1)) >> 11
  nit = (nch + 1) >> 1
  rot_key = (_iota() + (L - 1)) & (L - 1)

  def issue(ch, slot):
    st = pl.multiple_of(e0a + ch * CHUNK, 16)
    pltpu.async_copy(pk_hbm.at[pl.ds(st, CHUNK)],
                     pk_b.at[pl.ds(slot * CHUNK, CHUNK)], sem_pk.at[slot])
    pltpu.async_copy(ce_hbm.at[pl.ds(st, CHUNK)],
                     ce_b.at[pl.ds(slot * CHUNK, CHUNK)], sem_ce.at[slot])

  def wait(slot):
    pltpu.make_async_copy(pk_hbm.at[pl.ds(0, CHUNK)],
                          pk_b.at[pl.ds(slot * CHUNK, CHUNK)],
                          sem_pk.at[slot]).wait()
    pltpu.make_async_copy(ce_hbm.at[pl.ds(0, CHUNK)],
                          ce_b.at[pl.ds(slot * CHUNK, CHUNK)],
                          sem_ce.at[slot]).wait()

  def do_chunk(ch, slot):
    bc = e0a + ch * CHUNK

    def vec(v, _):
      o = slot * CHUNK + v * L
      pk = pk_b[pl.ds(o, L)]
      ce = ce_b[pl.ds(o, L)]
      ls = lax.shift_right_logical(pk, DST_BITS)
      dstv = lax.bitwise_and(pk, DST_MASK)
      pos = (bc + v * L) + _iota()
      m = (pos >= e0) & (pos < e1)
      ydst = plsc.load_gather(y_full, [dstv])
      ysrc = plsc.load_gather(y_full, [ls + node_lo])
      val = jnp.maximum(ce * (ysrc - ydst), 0.0)
      packed = (ls.astype(jnp.uint32) << jnp.uint32(20)) | (
          lax.shift_right_logical(
              lax.bitcast_convert_type(val, jnp.uint32), jnp.uint32(12)))
      packed = jnp.where(m, packed, jnp.uint32(0))
      cm = plsc.cummax(packed)
      segmax = lax.bitcast_convert_type(
          lax.bitwise_and(cm, jnp.uint32(0xFFFFF)) << jnp.uint32(12),
          jnp.float32)
      _, nls = plsc.sort_key_val(rot_key, ls)
      is_last = (ls != nls) | (pos + 1 >= e1) | (_iota() == (L - 1))
      cur = plsc.load_gather(gm_v, [ls])
      plsc.store_scatter(gm_v, [ls], jnp.maximum(cur, segmax),
                         mask=m & is_last)
      return 0

    lax.fori_loop(0, NVEC, vec, 0)

  issue(0, 0)
  issue(1, 1)

  def body(it, _):
    wait(0)
    do_chunk(2 * it, 0)
    issue(2 * it + 2, 0)
    wait(1)
    do_chunk(2 * it + 1, 1)
    issue(2 * it + 3, 1)
    return 0

  lax.fori_loop(0, nit, body, 0)
  wait(0)
  wait(1)


def _make_eval():
  def body(ytmp_in, ybase_in, acc_in, pk_hbm, ce_hbm, bnd_hbm, par_hbm,
           ytmp_out, acc_out, ybase_out,
           y_full, ybase_v, acc_v, gm_v, stage_v, ybst_v, bnd_v, par_v,
           pk_b, ce_b, sem_y, sem_c, sem_pk, sem_ce):
    wid = lax.axis_index("s") * NC + lax.axis_index("c")
    node_lo = pl.multiple_of(wid * NPT, NPT)

    cp_y = pltpu.make_async_copy(ytmp_in, y_full, sem_y)
    cp_y.start()
    pltpu.sync_copy(bnd_hbm, bnd_v)
    pltpu.sync_copy(par_hbm, par_v)
    pltpu.async_copy(ybase_in.at[pl.ds(node_lo, NPT)], ybase_v, sem_c).wait()
    pltpu.async_copy(acc_in.at[pl.ds(node_lo, NPT)], acc_v, sem_c).wait()
    wa = pl.multiple_of((wid >> 3) << 3, 8)
    ev = bnd_v[pl.ds(wa, L)]
    lane = wid - wa
    e0 = jnp.max(jnp.where(_iota() == lane, ev, 0))
    e1 = jnp.max(jnp.where(_iota() == lane + 1, ev, 0))
    pv = par_v[pl.ds(0, L)]
    accmul = pv[0]
    w = pv[1]
    s1 = pv[2]
    s2 = pv[3]
    b1 = pv[4]

    def zero(i, _):
      gm_v[pl.ds(i * L, L)] = jnp.zeros((L,), jnp.float32)
      return 0

    lax.fori_loop(0, NPT // L, zero, 0)
    cp_y.wait()

    _sweep(e0, e1, node_lo, pk_hbm, ce_hbm, y_full, gm_v, pk_b, ce_b,
           sem_pk, sem_ce)

    def combine(i, _):
      o = i * L
      k = 1.0 - gm_v[pl.ds(o, L)]
      yb = ybase_v[pl.ds(o, L)]
      a = accmul * acc_v[pl.ds(o, L)] + w * k
      acc_v[pl.ds(o, L)] = a
      upd = s1 * k + s2 * a
      stage_v[pl.ds(o, L)] = yb + upd
      ybst_v[pl.ds(o, L)] = yb + b1 * upd
      return 0

    lax.fori_loop(0, NPT // L, combine, 0)

    pltpu.async_copy(stage_v, ytmp_out.at[pl.ds(node_lo, NPT)], sem_c).wait()
    pltpu.async_copy(acc_v, acc_out.at[pl.ds(node_lo, NPT)], sem_c).wait()
    pltpu.async_copy(ybst_v, ybase_out.at[pl.ds(node_lo, NPT)], sem_c).wait()

  f32 = jnp.float32
  out_type = (jax.ShapeDtypeStruct((NP,), f32),
              jax.ShapeDtypeStruct((NP,), f32),
              jax.ShapeDtypeStruct((NP,), f32))
  return pl.kernel(
      body,
      out_type=out_type,
      mesh=plsc.VectorSubcoreMesh(core_axis_name="c", subcore_axis_name="s",
                                  num_cores=NC, num_subcores=NS),
      compiler_params=pltpu.CompilerParams(needs_layout_passes=False),
      scratch_types=[
          pltpu.VMEM((NP,), f32),
          pltpu.VMEM((NPT,), f32),
          pltpu.VMEM((NPT,), f32),
          pltpu.VMEM((NPT,), f32),
          pltpu.VMEM((NPT,), f32),
          pltpu.VMEM((NPT,), f32),
          pltpu.VMEM((128,), jnp.int32),
          pltpu.VMEM((128,), f32),
          pltpu.VMEM((2 * CHUNK,), jnp.int32),
          pltpu.VMEM((2 * CHUNK,), f32),
          pltpu.SemaphoreType.DMA,
          pltpu.SemaphoreType.DMA,
          pltpu.SemaphoreType.DMA((2,)),
          pltpu.SemaphoreType.DMA((2,)),
      ],
  )


_EV = _make_eval()


def _params():
  import numpy as _np
  a = _np.zeros((4, 128), _np.float32)
  a[0, :5] = (0.0, 1.0, H / 2, 0.0, 0.0)
  a[1, :5] = (1.0, 2.0, H / 2, 0.0, 0.0)
  a[2, :5] = (1.0, 2.0, H, 0.0, 0.0)
  a[3, :5] = (1.0, 1.0, 0.0, H / 6, 1.0)
  return jnp.asarray(a)


def kernel(x, edge_attr, mask, front_initial, edge_index):
  del x, mask
  src_s, dst_s, ea_s = lax.sort(
      [edge_index[0], edge_index[1], edge_attr], num_keys=1)
  deg = jax.ops.segment_sum(ea_s, src_s, num_segments=N,
                            indices_are_sorted=True)
  ce = jnp.sqrt(ea_s) / deg[src_s]
  ls = src_s - (src_s // NPT) * NPT
  pk = (ls << DST_BITS) | dst_s
  bnd = jnp.searchsorted(
      src_s, jnp.arange(33, dtype=jnp.int32) * NPT).astype(jnp.int32)
  bnd64 = jnp.zeros((128,), jnp.int32).at[:33].set(bnd)
  pk_p = jnp.concatenate([pk, jnp.zeros((PAD,), jnp.int32)])
  ce_p = jnp.concatenate([ce, jnp.zeros((PAD,), jnp.float32)])
  y0 = jnp.concatenate([front_initial[:, 0],
                        jnp.zeros((NP - N,), jnp.float32)])

  pall = _params()

  def stage(i, carry):
    yt, yb, ac = carry
    par = lax.dynamic_slice(pall, (lax.rem(i, 4), 0), (1, 128))[0]
    yt, ac, yb = _EV(yt, yb, ac, pk_p, ce_p, bnd64, par)
    return yt, yb, ac

  yt, yb, ac = lax.fori_loop(
      0, 40, stage, (y0, y0, jnp.zeros((NP,), jnp.float32)))
  return yb[:N].reshape(1, N)


# speedup vs baseline: 208.0730x; 3.1904x over previous
import jax
import jax.numpy as jnp
from jax import lax
from jax.experimental import pallas as pl
from jax.experimental.pallas import tpu as pltpu
from jax.experimental.pallas import tpu_sc as plsc

N = 100000
E = 3200000
NC, NS, L = 2, 16, 16
NW = NC * NS
NPT = 3136
NP = NPT * NW
CHUNK = 2048
NVEC = CHUNK // L
PAD = 4 * CHUNK
EP = E + PAD
H = 0.1
DST_BITS = 17
DST_MASK = (1 << DST_BITS) - 1


def _iota():
  return lax.iota(jnp.int32, L)


def _sweep(e0, e1, node_lo, pk_hbm, ce_hbm, y_full, gm_v, ivd_v, pk_b, ce_b,
           sem_pk, sem_ce):
  e0a = (e0 >> 4) << 4
  nch = (e1 - e0a + (CHUNK - 1)) >> 11
  nit = (nch + 1) >> 1
  rot_key = (_iota() + (L - 1)) & (L - 1)

  def issue(ch, slot):
    st = pl.multiple_of(e0a + ch * CHUNK, 16)
    pltpu.async_copy(pk_hbm.at[pl.ds(st, CHUNK)],
                     pk_b.at[pl.ds(slot * CHUNK, CHUNK)], sem_pk.at[slot])
    pltpu.async_copy(ce_hbm.at[pl.ds(st, CHUNK)],
                     ce_b.at[pl.ds(slot * CHUNK, CHUNK)], sem_ce.at[slot])

  def wait(slot):
    pltpu.make_async_copy(pk_hbm.at[pl.ds(0, CHUNK)],
                          pk_b.at[pl.ds(slot * CHUNK, CHUNK)],
                          sem_pk.at[slot]).wait()
    pltpu.make_async_copy(ce_hbm.at[pl.ds(0, CHUNK)],
                          ce_b.at[pl.ds(slot * CHUNK, CHUNK)],
                          sem_ce.at[slot]).wait()

  def do_chunk(ch, slot):
    bc = e0a + ch * CHUNK

    def vec(v, _):
      o = slot * CHUNK + v * L
      pk = pk_b[pl.ds(o, L)]
      ce = ce_b[pl.ds(o, L)]
      ls = lax.shift_right_logical(pk, DST_BITS)
      dstv = lax.bitwise_and(pk, DST_MASK)
      pos = (bc + v * L) + _iota()
      m = (pos >= e0) & (pos < e1)
      ydst = plsc.load_gather(y_full, [dstv])
      ysrc = plsc.load_gather(y_full, [ls + node_lo])
      iv = plsc.load_gather(ivd_v, [ls])
      val = ce * jnp.maximum((ysrc - ydst) * iv, 0.0)
      packed = (ls.astype(jnp.uint32) << jnp.uint32(20)) | (
          lax.shift_right_logical(
              lax.bitcast_convert_type(val, jnp.uint32), jnp.uint32(12)))
      packed = jnp.where(m, packed, jnp.uint32(0))
      cm = plsc.cummax(packed)
      segmax = lax.bitcast_convert_type(
          lax.bitwise_and(cm, jnp.uint32(0xFFFFF)) << jnp.uint32(12),
          jnp.float32)
      _, nls = plsc.sort_key_val(rot_key, ls)
      is_last = (ls != nls) | (pos + 1 >= e1) | (_iota() == (L - 1))
      cur = plsc.load_gather(gm_v, [ls])
      plsc.store_scatter(gm_v, [ls], jnp.maximum(cur, segmax),
                         mask=m & is_last)
      return 0

    lax.fori_loop(0, NVEC, vec, 0, unroll=4)

  issue(0, 0)
  issue(1, 1)

  def body(it, _):
    wait(0)
    do_chunk(2 * it, 0)
    issue(2 * it + 2, 0)
    wait(1)
    do_chunk(2 * it + 1, 1)
    issue(2 * it + 3, 1)
    return 0

  lax.fori_loop(0, nit, body, 0)
  wait(0)
  wait(1)


def _make_eval():
  def body(ytmp_in, ybase_in, acc_in, pk_hbm, ce_hbm, bnd_hbm, par_hbm,
           ivd_hbm, ytmp_out, acc_out, ybase_out,
           y_full, ybase_v, acc_v, gm_v, stage_v, ybst_v, bnd_v, par_v,
           ivd_v, pk_b, ce_b, sem_y, sem_c, sem_pk, sem_ce):
    wid = lax.axis_index("s") * NC + lax.axis_index("c")
    node_lo = pl.multiple_of(wid * NPT, NPT)

    cp_y = pltpu.make_async_copy(ytmp_in, y_full, sem_y)
    cp_y.start()
    pltpu.sync_copy(bnd_hbm, bnd_v)
    pltpu.sync_copy(par_hbm, par_v)
    pltpu.async_copy(ybase_in.at[pl.ds(node_lo, NPT)], ybase_v, sem_c).wait()
    pltpu.async_copy(acc_in.at[pl.ds(node_lo, NPT)], acc_v, sem_c).wait()
    pltpu.async_copy(ivd_hbm.at[pl.ds(node_lo, NPT)], ivd_v, sem_c).wait()
    wa = pl.multiple_of((wid >> 3) << 3, 8)
    ev = bnd_v[pl.ds(wa, L)]
    lane = wid - wa
    e0 = jnp.max(jnp.where(_iota() == lane, ev, 0))
    e1 = jnp.max(jnp.where(_iota() == lane + 1, ev, 0))
    pv = par_v[pl.ds(0, L)]
    accmul = pv[0]
    w = pv[1]
    s1 = pv[2]
    s2 = pv[3]
    b1 = pv[4]

    def zero(i, _):
      gm_v[pl.ds(i * L, L)] = jnp.zeros((L,), jnp.float32)
      return 0

    lax.fori_loop(0, NPT // L, zero, 0)
    cp_y.wait()

    _sweep(e0, e1, node_lo, pk_hbm, ce_hbm, y_full, gm_v, ivd_v, pk_b, ce_b,
           sem_pk, sem_ce)

    def combine(i, _):
      o = i * L
      k = 1.0 - gm_v[pl.ds(o, L)]
      yb = ybase_v[pl.ds(o, L)]
      a = accmul * acc_v[pl.ds(o, L)] + w * k
      acc_v[pl.ds(o, L)] = a
      upd = s1 * k + s2 * a
      stage_v[pl.ds(o, L)] = yb + upd
      ybst_v[pl.ds(o, L)] = yb + b1 * upd
      return 0

    lax.fori_loop(0, NPT // L, combine, 0)

    pltpu.async_copy(stage_v, ytmp_out.at[pl.ds(node_lo, NPT)], sem_c).wait()
    pltpu.async_copy(acc_v, acc_out.at[pl.ds(node_lo, NPT)], sem_c).wait()
    pltpu.async_copy(ybst_v, ybase_out.at[pl.ds(node_lo, NPT)], sem_c).wait()

  f32 = jnp.float32
  out_type = (jax.ShapeDtypeStruct((NP,), f32),
              jax.ShapeDtypeStruct((NP,), f32),
              jax.ShapeDtypeStruct((NP,), f32))
  return pl.kernel(
      body,
      out_type=out_type,
      mesh=plsc.VectorSubcoreMesh(core_axis_name="c", subcore_axis_name="s",
                                  num_cores=NC, num_subcores=NS),
      compiler_params=pltpu.CompilerParams(needs_layout_passes=False),
      scratch_types=[
          pltpu.VMEM((NP,), f32),
          pltpu.VMEM((NPT,), f32),
          pltpu.VMEM((NPT,), f32),
          pltpu.VMEM((NPT,), f32),
          pltpu.VMEM((NPT,), f32),
          pltpu.VMEM((NPT,), f32),
          pltpu.VMEM((128,), jnp.int32),
          pltpu.VMEM((128,), f32),
          pltpu.VMEM((NPT,), f32),
          pltpu.VMEM((2 * CHUNK,), jnp.int32),
          pltpu.VMEM((2 * CHUNK,), f32),
          pltpu.SemaphoreType.DMA,
          pltpu.SemaphoreType.DMA,
          pltpu.SemaphoreType.DMA((2,)),
          pltpu.SemaphoreType.DMA((2,)),
      ],
  )


_EV = _make_eval()


def _params():
  import numpy as _np
  a = _np.zeros((4, 128), _np.float32)
  a[0, :5] = (0.0, 1.0, H / 2, 0.0, 0.0)
  a[1, :5] = (1.0, 2.0, H / 2, 0.0, 0.0)
  a[2, :5] = (1.0, 2.0, H, 0.0, 0.0)
  a[3, :5] = (1.0, 1.0, 0.0, H / 6, 1.0)
  return jnp.asarray(a)


def kernel(x, edge_attr, mask, front_initial, edge_index):
  del x, mask
  src_s, dst_s, ea_s = lax.sort(
      [edge_index[0], edge_index[1], edge_attr], num_keys=1)
  deg = jax.ops.segment_sum(ea_s, src_s, num_segments=N,
                            indices_are_sorted=True)
  ivd = jnp.concatenate([1.0 / deg, jnp.zeros((NP - N,), jnp.float32)])
  ce = jnp.sqrt(ea_s)
  ls = src_s - (src_s // NPT) * NPT
  pk = (ls << DST_BITS) | dst_s
  bnd = jnp.searchsorted(
      src_s, jnp.arange(33, dtype=jnp.int32) * NPT).astype(jnp.int32)
  bnd64 = jnp.zeros((128,), jnp.int32).at[:33].set(bnd)
  pk_p = jnp.concatenate([pk, jnp.zeros((PAD,), jnp.int32)])
  ce_p = jnp.concatenate([ce, jnp.zeros((PAD,), jnp.float32)])
  y0 = jnp.concatenate([front_initial[:, 0],
                        jnp.zeros((NP - N,), jnp.float32)])

  pall = _params()

  def stage(i, carry):
    yt, yb, ac = carry
    par = lax.dynamic_slice(pall, (lax.rem(i, 4), 0), (1, 128))[0]
    yt, ac, yb = _EV(yt, yb, ac, pk_p, ce_p, bnd64, par, ivd)
    return yt, yb, ac

  yt, yb, ac = lax.fori_loop(
      0, 40, stage, (y0, y0, jnp.zeros((NP,), jnp.float32)))
  return yb[:N].reshape(1, N)


# speedup vs baseline: 208.7277x; 1.0031x over previous
import jax
import jax.numpy as jnp
from jax import lax
from jax.experimental import pallas as pl
from jax.experimental.pallas import tpu as pltpu
from jax.experimental.pallas import tpu_sc as plsc

N = 100000
E = 3200000
NC, NS, L = 2, 16, 16
NW = NC * NS
NPT = 3136
NP = NPT * NW
CHUNK = 2048
NVEC = CHUNK // L
PAD = 4 * CHUNK
EP = E + PAD
H = 0.1
DST_BITS = 17
DST_MASK = (1 << DST_BITS) - 1


def _iota():
  return lax.iota(jnp.int32, L)


def _sweep(e0, e1, node_lo, pk_hbm, ce_hbm, y_full, gm_v, ivd_v, pk_b, ce_b,
           sem_pk, sem_ce):
  e0a = (e0 >> 4) << 4
  nch = (e1 - e0a + (CHUNK - 1)) >> 11
  nit = (nch + 1) >> 1
  rot_key = (_iota() + (L - 1)) & (L - 1)

  def issue(ch, slot):
    st = pl.multiple_of(e0a + ch * CHUNK, 16)
    pltpu.async_copy(pk_hbm.at[pl.ds(st, CHUNK)],
                     pk_b.at[pl.ds(slot * CHUNK, CHUNK)], sem_pk.at[slot])
    pltpu.async_copy(ce_hbm.at[pl.ds(st, CHUNK)],
                     ce_b.at[pl.ds(slot * CHUNK, CHUNK)], sem_ce.at[slot])

  def wait(slot):
    pltpu.make_async_copy(pk_hbm.at[pl.ds(0, CHUNK)],
                          pk_b.at[pl.ds(slot * CHUNK, CHUNK)],
                          sem_pk.at[slot]).wait()
    pltpu.make_async_copy(ce_hbm.at[pl.ds(0, CHUNK)],
                          ce_b.at[pl.ds(slot * CHUNK, CHUNK)],
                          sem_ce.at[slot]).wait()

  def do_chunk(ch, slot):
    bc = e0a + ch * CHUNK

    def vec(v, _):
      o = slot * CHUNK + v * L
      pk = pk_b[pl.ds(o, L)]
      ce = ce_b[pl.ds(o, L)]
      ls = lax.shift_right_logical(pk, DST_BITS)
      dstv = lax.bitwise_and(pk, DST_MASK)
      pos = (bc + v * L) + _iota()
      m = (pos >= e0) & (pos < e1)
      ydst = plsc.load_gather(y_full, [dstv])
      ysrc = plsc.load_gather(y_full, [ls + node_lo])
      iv = plsc.load_gather(ivd_v, [ls])
      val = ce * jnp.maximum((ysrc - ydst) * iv, 0.0)
      packed = (ls.astype(jnp.uint32) << jnp.uint32(20)) | (
          lax.shift_right_logical(
              lax.bitcast_convert_type(val, jnp.uint32), jnp.uint32(12)))
      packed = jnp.where(m, packed, jnp.uint32(0))
      cm = plsc.cummax(packed)
      segmax = lax.bitcast_convert_type(
          lax.bitwise_and(cm, jnp.uint32(0xFFFFF)) << jnp.uint32(12),
          jnp.float32)
      pk1 = pk_b[pl.ds(o + 1, L)]
      nls = lax.shift_right_logical(pk1, DST_BITS)
      is_last = (ls != nls) | (pos + 1 >= e1) | (_iota() == (L - 1))
      cur = plsc.load_gather(gm_v, [ls])
      plsc.store_scatter(gm_v, [ls], jnp.maximum(cur, segmax),
                         mask=m & is_last)
      return 0

    lax.fori_loop(0, NVEC, vec, 0, unroll=4)

  issue(0, 0)
  issue(1, 1)

  def body(it, _):
    wait(0)
    do_chunk(2 * it, 0)
    issue(2 * it + 2, 0)
    wait(1)
    do_chunk(2 * it + 1, 1)
    issue(2 * it + 3, 1)
    return 0

  lax.fori_loop(0, nit, body, 0)
  wait(0)
  wait(1)


def _make_eval():
  def body(ytmp_in, ybase_in, acc_in, pk_hbm, ce_hbm, bnd_hbm, par_hbm,
           ivd_hbm, ytmp_out, acc_out, ybase_out,
           y_full, ybase_v, acc_v, gm_v, stage_v, ybst_v, bnd_v, par_v,
           ivd_v, pk_b, ce_b, sem_y, sem_c, sem_pk, sem_ce):
    wid = lax.axis_index("s") * NC + lax.axis_index("c")
    node_lo = pl.multiple_of(wid * NPT, NPT)

    cp_y = pltpu.make_async_copy(ytmp_in, y_full, sem_y)
    cp_y.start()
    pltpu.sync_copy(bnd_hbm, bnd_v)
    pltpu.sync_copy(par_hbm, par_v)
    pltpu.async_copy(ybase_in.at[pl.ds(node_lo, NPT)], ybase_v, sem_c).wait()
    pltpu.async_copy(acc_in.at[pl.ds(node_lo, NPT)], acc_v, sem_c).wait()
    pltpu.async_copy(ivd_hbm.at[pl.ds(node_lo, NPT)], ivd_v, sem_c).wait()
    wa = pl.multiple_of((wid >> 3) << 3, 8)
    ev = bnd_v[pl.ds(wa, L)]
    lane = wid - wa
    e0 = jnp.max(jnp.where(_iota() == lane, ev, 0))
    e1 = jnp.max(jnp.where(_iota() == lane + 1, ev, 0))
    pv = par_v[pl.ds(0, L)]
    accmul = pv[0]
    w = pv[1]
    s1 = pv[2]
    s2 = pv[3]
    b1 = pv[4]

    def zero(i, _):
      gm_v[pl.ds(i * L, L)] = jnp.zeros((L,), jnp.float32)
      return 0

    lax.fori_loop(0, NPT // L, zero, 0)
    cp_y.wait()

    _sweep(e0, e1, node_lo, pk_hbm, ce_hbm, y_full, gm_v, ivd_v, pk_b, ce_b,
           sem_pk, sem_ce)

    def combine(i, _):
      o = i * L
      k = 1.0 - gm_v[pl.ds(o, L)]
      yb = ybase_v[pl.ds(o, L)]
      a = accmul * acc_v[pl.ds(o, L)] + w * k
      acc_v[pl.ds(o, L)] = a
      upd = s1 * k + s2 * a
      stage_v[pl.ds(o, L)] = yb + upd
      ybst_v[pl.ds(o, L)] = yb + b1 * upd
      return 0

    lax.fori_loop(0, NPT // L, combine, 0)

    pltpu.async_copy(stage_v, ytmp_out.at[pl.ds(node_lo, NPT)], sem_c).wait()
    pltpu.async_copy(acc_v, acc_out.at[pl.ds(node_lo, NPT)], sem_c).wait()
    pltpu.async_copy(ybst_v, ybase_out.at[pl.ds(node_lo, NPT)], sem_c).wait()

  f32 = jnp.float32
  out_type = (jax.ShapeDtypeStruct((NP,), f32),
              jax.ShapeDtypeStruct((NP,), f32),
              jax.ShapeDtypeStruct((NP,), f32))
  return pl.kernel(
      body,
      out_type=out_type,
      mesh=plsc.VectorSubcoreMesh(core_axis_name="c", subcore_axis_name="s",
                                  num_cores=NC, num_subcores=NS),
      compiler_params=pltpu.CompilerParams(needs_layout_passes=False),
      scratch_types=[
          pltpu.VMEM((NP,), f32),
          pltpu.VMEM((NPT,), f32),
          pltpu.VMEM((NPT,), f32),
          pltpu.VMEM((NPT,), f32),
          pltpu.VMEM((NPT,), f32),
          pltpu.VMEM((NPT,), f32),
          pltpu.VMEM((128,), jnp.int32),
          pltpu.VMEM((128,), f32),
          pltpu.VMEM((NPT,), f32),
          pltpu.VMEM((2 * CHUNK + L,), jnp.int32),
          pltpu.VMEM((2 * CHUNK,), f32),
          pltpu.SemaphoreType.DMA,
          pltpu.SemaphoreType.DMA,
          pltpu.SemaphoreType.DMA((2,)),
          pltpu.SemaphoreType.DMA((2,)),
      ],
  )


_EV = _make_eval()


def _params():
  import numpy as _np
  a = _np.zeros((4, 128), _np.float32)
  a[0, :5] = (0.0, 1.0, H / 2, 0.0, 0.0)
  a[1, :5] = (1.0, 2.0, H / 2, 0.0, 0.0)
  a[2, :5] = (1.0, 2.0, H, 0.0, 0.0)
  a[3, :5] = (1.0, 1.0, 0.0, H / 6, 1.0)
  return jnp.asarray(a)


def kernel(x, edge_attr, mask, front_initial, edge_index):
  del x, mask
  src_s, dst_s, ea_s = lax.sort(
      [edge_index[0], edge_index[1], edge_attr], num_keys=1)
  deg = jax.ops.segment_sum(ea_s, src_s, num_segments=N,
                            indices_are_sorted=True)
  ivd = jnp.concatenate([1.0 / deg, jnp.zeros((NP - N,), jnp.float32)])
  ce = jnp.sqrt(ea_s)
  ls = src_s - (src_s // NPT) * NPT
  pk = (ls << DST_BITS) | dst_s
  bnd = jnp.searchsorted(
      src_s, jnp.arange(33, dtype=jnp.int32) * NPT).astype(jnp.int32)
  bnd64 = jnp.zeros((128,), jnp.int32).at[:33].set(bnd)
  pk_p = jnp.concatenate([pk, jnp.zeros((PAD,), jnp.int32)])
  ce_p = jnp.concatenate([ce, jnp.zeros((PAD,), jnp.float32)])
  y0 = jnp.concatenate([front_initial[:, 0],
                        jnp.zeros((NP - N,), jnp.float32)])

  pall = _params()

  def stage(i, carry):
    yt, yb, ac = carry
    par = lax.dynamic_slice(pall, (lax.rem(i, 4), 0), (1, 128))[0]
    yt, ac, yb = _EV(yt, yb, ac, pk_p, ce_p, bnd64, par, ivd)
    return yt, yb, ac

  yt, yb, ac = lax.fori_loop(
      0, 40, stage, (y0, y0, jnp.zeros((NP,), jnp.float32)))
  return yb[:N].reshape(1, N)
